# baseline (speedup 1.0000x reference)
import jax
import jax.numpy as jnp
from jax.experimental import pallas as pl

_C = 4
_M = 8192
_D = 2048
_B = 64
_BETA = 0.05
_BG_KNN = 50
_TILE = 1024


def _sims_body(f_ref, v_ref, o_ref):
    o_ref[...] = jax.lax.dot_general(
        f_ref[...], v_ref[...],
        dimension_numbers=(((1,), (1,)), ((), ())),
        preferred_element_type=jnp.float32,
    )


def _compute_sims(features, tempV):
    n = _C * _M
    return pl.pallas_call(
        _sims_body,
        grid=(n // _TILE,),
        in_specs=[
            pl.BlockSpec((_B, _D), lambda i: (0, 0)),
            pl.BlockSpec((_TILE, _D), lambda i: (i, 0)),
        ],
        out_specs=pl.BlockSpec((_B, _TILE), lambda i: (0, i)),
        out_shape=jax.ShapeDtypeStruct((_B, n), jnp.float32),
    )(features, tempV)


def kernel(features, percam_memory, cams, plabel, proxy):
    tempV = percam_memory.reshape(_C * _M, _D)
    sims = _compute_sims(features, tempV)
    target_inputs = sims / _BETA
    rows = jnp.arange(_B)
    counts = jnp.bincount(cams, length=_C)
    n_k = jnp.maximum(counts[cams], 1).astype(jnp.float32)

    mapped = proxy - cams * _M

    cam_cols = cams[:, None] * _M + jnp.arange(_M)[None, :]
    logits1 = jnp.take_along_axis(target_inputs, cam_cols, axis=1)
    logp1 = jax.nn.log_softmax(logits1, axis=1)
    ce = -jnp.take_along_axis(logp1, mapped[:, None], axis=1)[:, 0]
    loss1 = jnp.sum(0.6 * ce / n_k)

    ori = plabel[:, None] + jnp.arange(_C)[None, :] * _M
    offline = sims.at[rows[:, None], ori].set(-10000.0)
    _, sel = jax.lax.top_k(offline, _BG_KNN)
    ti_ori = jnp.take_along_axis(target_inputs, ori, axis=1)
    ti_sel = jnp.take_along_axis(target_inputs, sel, axis=1)
    concat = jnp.concatenate([ti_ori, ti_sel], axis=1)
    tgt = jnp.concatenate(
        [jnp.full((_B, _C), 1.0 / _C, dtype=jnp.float32),
         jnp.zeros((_B, _BG_KNN), dtype=jnp.float32)], axis=1)
    l2 = -(jax.nn.log_softmax(concat, axis=1) * tgt).sum(axis=1)
    loss2 = jnp.sum(0.7 * l2 / n_k)

    percam_sims = sims.reshape(_B, _C, _M)
    cam_tops = jnp.argmax(percam_sims, axis=2) + jnp.arange(_C)[None, :] * _M
    top_vals = jnp.take_along_axis(sims, cam_tops, axis=1)
    _, sel3 = jax.lax.top_k(top_vals, 3)
    pos_tops = jnp.take_along_axis(cam_tops, sel3, axis=1)
    online = sims.at[rows[:, None], pos_tops].set(10000.0)
    _, top_inds = jax.lax.top_k(online, 30 + 3)
    sel_input = jnp.take_along_axis(target_inputs, top_inds, axis=1)
    tgt3 = jnp.concatenate(
        [jnp.full((_B, 3), 1.0 / 3.0, dtype=jnp.float32),
         jnp.zeros((_B, 30), dtype=jnp.float32)], axis=1)
    l3 = -(jax.nn.log_softmax(sel_input, axis=1) * tgt3).sum(axis=1)
    loss3 = jnp.sum(0.7 * l3 / n_k)

    return jnp.reshape(loss1 + loss2 + loss3, (1,))
